# baseline (speedup 1.0000x reference)
import jax
import jax.numpy as jnp
from jax import lax
from jax.experimental import pallas as pl
from jax.experimental.pallas import tpu as pltpu
from jax.experimental.pallas import tpu_sc as plsc

N_NODES = 100000
N_EDGES = 6400000

NW = 32
NBLK = N_EDGES // 128
BLK_PW = NBLK // NW
NEXTRA = NBLK - NW * BLK_PW
CB = 11
NCHUNK = BLK_PW // CB
B = CB * 128
NG = B // 16

_HALF = 0.5
_THREEHALF = 1.5
_MAGIC = 0x5F3759DF

_NSTAGE = 10
_SSLICE = N_NODES // _NSTAGE


def _norm16(dx, dy, dz):
    sq = dx * dx + dy * dy + dz * dz
    sqm = jnp.maximum(sq, 1e-35)
    magic = jnp.full((16,), _MAGIC, jnp.int32)
    y = lax.bitcast_convert_type(
        magic - (lax.bitcast_convert_type(sqm, jnp.int32) >> 1), jnp.float32)
    y = y * (_THREEHALF - _HALF * sqm * y * y)
    y = y * (_THREEHALF - _HALF * sqm * y * y)
    return sq * y


def _compute(nblk, eb, gx, gy, gz, vec_v, w_v):
    def body(b, _):
        b256 = b * 256
        b128 = b * 128
        for g in range(8):
            o = g * 16
            s2 = pl.ds(b256 + o, 16)
            d2 = pl.ds(b256 + 128 + o, 16)
            dx = gx[s2] - gx[d2]
            dy = gy[s2] - gy[d2]
            dz = gz[s2] - gz[d2]
            w_v[pl.ds(b128 + o, 16)] = _norm16(dx, dy, dz)
            vec_v[b, 0, pl.ds(o, 16)] = dx
            vec_v[b, 1, pl.ds(o, 16)] = dy
            vec_v[b, 2, pl.ds(o, 16)] = dz
        return 0

    lax.fori_loop(0, nblk, body, 0)


def _sc_body(pos_hbm, edge_hbm, w_hbm, vec_hbm,
             sx, sy, sz,
             eb0, eb1, g0, g1, vec0, vec1, w0, w1,
             stage_v, gsem0, gsem1, osem0, osem1):
    cid = lax.axis_index("c")
    sid = lax.axis_index("s")
    wid = cid * 16 + sid

    for c, comp in enumerate((sx, sy, sz)):
        for j in range(_NSTAGE):
            t = c * _NSTAGE + j

            @pl.when(sid == t % 16)
            def _(c=c, comp=comp, j=j):
                off = j * _SSLICE
                pltpu.sync_copy(
                    pos_hbm.at[pl.ds(c * N_NODES + off, _SSLICE)], stage_v)
                pltpu.sync_copy(stage_v, comp.at[pl.ds(off, _SSLICE)])

    plsc.subcore_barrier()

    base_blk = wid * BLK_PW + jnp.minimum(wid, NEXTRA)
    bufs = ((eb0, g0[0], g0[1], g0[2], vec0, w0, gsem0, osem0),
            (eb1, g1[0], g1[1], g1[2], vec1, w1, gsem1, osem1))

    def load_idx(k, eb):
        off = (base_blk + k * CB) * 256
        pltpu.sync_copy(edge_hbm.at[pl.ds(off, CB * 256)], eb)

    def fire_gathers(eb, gx, gy, gz, gsem):
        for comp, dst in zip((sx, sy, sz), (gx, gy, gz)):
            pltpu.async_copy(comp.at[eb], dst, gsem)

    def wait_gathers(eb, gx, gy, gz, gsem):
        for comp, dst in zip((sx, sy, sz), (gx, gy, gz)):
            pltpu.make_async_copy(comp.at[eb], dst, gsem).wait()

    def fire_out(k, vec_v, w_v, osem):
        blk = base_blk + k * CB
        pltpu.async_copy(vec_v, vec_hbm.at[pl.ds(blk, CB)], osem)
        pltpu.async_copy(w_v, w_hbm.at[pl.ds(blk * 128, B)], osem)

    def wait_out(k, vec_v, w_v, osem):
        blk = base_blk + k * CB
        pltpu.make_async_copy(
            vec_v, vec_hbm.at[pl.ds(blk, CB)], osem).wait()
        pltpu.make_async_copy(
            w_v, w_hbm.at[pl.ds(blk * 128, B)], osem).wait()

    load_idx(0, eb0)
    fire_gathers(eb0, g0[0], g0[1], g0[2], gsem0)

    def outer(ki, _):
        for h in (0, 1):
            k = 2 * ki + h
            eb, gx, gy, gz, vec_v, w_v, gsem, osem = bufs[h]
            neb, ngx, ngy, ngz, _nv, _nw, ngsem, _no = bufs[1 - h]

            wait_gathers(eb, gx, gy, gz, gsem)

            @pl.when(k + 1 < NCHUNK)
            def _():
                load_idx(k + 1, neb)
                fire_gathers(neb, ngx, ngy, ngz, ngsem)

            @pl.when(ki >= 1)
            def _():
                wait_out(k, vec_v, w_v, osem)

            _compute(CB, eb, gx, gy, gz, vec_v, w_v)
            fire_out(k, vec_v, w_v, osem)
        return 0

    lax.fori_loop(0, NCHUNK // 2, outer, 0)

    for h in (0, 1):
        eb, gx, gy, gz, vec_v, w_v, gsem, osem = bufs[h]
        wait_out(0, vec_v, w_v, osem)

    @pl.when(wid < NEXTRA)
    def _():
        eb, gx, gy, gz, vec_v, w_v, gsem, osem = bufs[0]
        xblk = base_blk + BLK_PW
        pltpu.sync_copy(edge_hbm.at[pl.ds(xblk * 256, 256)],
                        eb.at[pl.ds(0, 256)])
        for comp, dst in zip((sx, sy, sz), (gx, gy, gz)):
            pltpu.sync_copy(comp.at[eb.at[pl.ds(0, 256)]],
                            dst.at[pl.ds(0, 256)])
        _compute(1, eb, gx, gy, gz, vec_v, w_v)
        pltpu.sync_copy(vec_v.at[pl.ds(0, 1)],
                        vec_hbm.at[pl.ds(xblk, 1)])
        pltpu.sync_copy(w_v.at[pl.ds(0, 128)],
                        w_hbm.at[pl.ds(xblk * 128, 128)])


@jax.jit
def _distance_sc(pos_flat, edge_flat):
    mesh = plsc.VectorSubcoreMesh(core_axis_name="c", subcore_axis_name="s")
    kfn = pl.kernel(
        _sc_body,
        out_type=[
            jax.ShapeDtypeStruct((N_EDGES,), jnp.float32),
            jax.ShapeDtypeStruct((NBLK, 4, 128), jnp.float32),
        ],
        mesh=mesh,
        compiler_params=pltpu.CompilerParams(needs_layout_passes=False),
        scratch_types=[
            pltpu.VMEM_SHARED((N_NODES,), jnp.float32),
            pltpu.VMEM_SHARED((N_NODES,), jnp.float32),
            pltpu.VMEM_SHARED((N_NODES,), jnp.float32),
            pltpu.VMEM((CB * 256,), jnp.int32),
            pltpu.VMEM((CB * 256,), jnp.int32),
            [pltpu.VMEM((CB * 256,), jnp.float32)] * 3,
            [pltpu.VMEM((CB * 256,), jnp.float32)] * 3,
            pltpu.VMEM((CB, 4, 128), jnp.float32),
            pltpu.VMEM((CB, 4, 128), jnp.float32),
            pltpu.VMEM((B,), jnp.float32),
            pltpu.VMEM((B,), jnp.float32),
            pltpu.VMEM((_SSLICE,), jnp.float32),
            pltpu.SemaphoreType.DMA,
            pltpu.SemaphoreType.DMA,
            pltpu.SemaphoreType.DMA,
            pltpu.SemaphoreType.DMA,
        ],
    )
    return kfn(pos_flat, edge_flat)


def kernel(pos, edge_index):
    pos_flat = pos.T.reshape(3 * N_NODES)
    edge_flat = (edge_index.reshape(2, NBLK, 128)
                 .transpose(1, 0, 2).reshape(2 * N_EDGES))
    edge_weight, vec_blk = _distance_sc(pos_flat, edge_flat)
    edge_vec = vec_blk[:, :3, :].transpose(0, 2, 1).reshape(N_EDGES, 3)
    return (edge_index, edge_weight, edge_vec)
